# baseline (speedup 1.0000x reference)
import jax
import jax.numpy as jnp
from jax.experimental import pallas as pl


def _copy_body(x_ref, o_ref):
    o_ref[...] = x_ref[...]


def _pallas_copy_rows(x, block_rows):
    n, m = x.shape
    assert n % block_rows == 0, (n, block_rows)
    return pl.pallas_call(
        _copy_body,
        grid=(n // block_rows,),
        in_specs=[pl.BlockSpec((block_rows, m), lambda i: (i, 0))],
        out_specs=pl.BlockSpec((block_rows, m), lambda i: (i, 0)),
        out_shape=jax.ShapeDtypeStruct(x.shape, x.dtype),
    )(x)


def _pallas_copy_whole(x):
    return pl.pallas_call(
        _copy_body,
        out_shape=jax.ShapeDtypeStruct(x.shape, x.dtype),
    )(x)


def kernel(node_features, edge_features, edges, node_hidden, edge_hidden,
           batch_indices, W1, W2, W3, U1, U2):
    nf = _pallas_copy_rows(node_features, 2000)
    ef = _pallas_copy_rows(edge_features, 16000)
    eg = _pallas_copy_whole(edges)
    nh = _pallas_copy_rows(node_hidden, 2000)
    eh = _pallas_copy_rows(edge_hidden, 4000)
    bi = _pallas_copy_whole(batch_indices.reshape(1250, 8)).reshape(10000)
    return (nf, ef, eg, nh, eh, bi)
